# Initial kernel scaffold; baseline (speedup 1.0000x reference)
import jax
import jax.numpy as jnp
from jax.experimental import pallas as pl


def kernel(pcs):
    raise NotImplementedError("write your pallas kernel here")


# speedup vs baseline: 4.8304x; 4.8304x over previous
import jax
import jax.numpy as jnp
from jax import lax
from jax.experimental import pallas as pl
from jax.experimental.pallas import tpu as pltpu
from jax.experimental.pallas import tpu_sc as plsc

B = 16
N = 16384
K = 2048
L = 16
NCHUNK = N // L


def _fps_body(pcs_hbm, out_hbm, px, py, pz, dist, outv):
    c = lax.axis_index("c")
    s = lax.axis_index("s")
    batch = c * 8 + s

    @pl.when(s < 8)
    def _():
        pltpu.sync_copy(pcs_hbm.at[pl.ds((batch * 3 + 0) * N, N)], px)
        pltpu.sync_copy(pcs_hbm.at[pl.ds((batch * 3 + 1) * N, N)], py)
        pltpu.sync_copy(pcs_hbm.at[pl.ds((batch * 3 + 2) * N, N)], pz)

        iota = lax.iota(jnp.int32, L)

        def init_body(j, carry):
            dist[pl.ds(j * L, L)] = jnp.full((L,), 1e10, jnp.float32)
            return carry

        lax.fori_loop(0, NCHUNK, init_body, 0)

        def step(i, carry):
            last, acc = carry
            lidx = jnp.full((L,), last, jnp.int32)
            lx = plsc.load_gather(px, [lidx])
            ly = plsc.load_gather(py, [lidx])
            lz = plsc.load_gather(pz, [lidx])

            def inner(j, carry):
                mv, mi = carry
                off = j * L
                dx = px[pl.ds(off, L)] - lx
                dy = py[pl.ds(off, L)] - ly
                dz = pz[pl.ds(off, L)] - lz
                d = dx * dx + dy * dy + dz * dz
                nd = jnp.minimum(dist[pl.ds(off, L)], d)
                dist[pl.ds(off, L)] = nd
                pred = nd > mv
                mv = jnp.where(pred, nd, mv)
                mi = jnp.where(pred, off + iota, mi)
                return mv, mi

            mv0 = jnp.full((L,), -1.0, jnp.float32)
            mi0 = jnp.zeros((L,), jnp.int32)
            mv, mi = lax.fori_loop(0, NCHUNK, inner, (mv0, mi0))
            m = jnp.max(mv)
            cand = jnp.where(mv == m, mi, jnp.int32(N))
            best = jnp.min(cand)
            acc = jnp.where(iota == (i & 15), jnp.full((L,), best, jnp.int32), acc)

            @pl.when((i & 15) == 15)
            def _():
                outv[pl.ds((i >> 4) * L, L)] = acc

            return best, acc

        lax.fori_loop(1, K, step, (jnp.int32(0), jnp.zeros((L,), jnp.int32)))
        pltpu.sync_copy(outv, out_hbm.at[pl.ds(batch * K, K)])


def kernel(pcs):
    mesh = plsc.VectorSubcoreMesh(core_axis_name="c", subcore_axis_name="s")
    out = pl.kernel(
        _fps_body,
        out_type=jax.ShapeDtypeStruct((B * K,), jnp.int32),
        mesh=mesh,
        compiler_params=pltpu.CompilerParams(needs_layout_passes=False),
        scratch_types=[
            pltpu.VMEM((N,), jnp.float32),
            pltpu.VMEM((N,), jnp.float32),
            pltpu.VMEM((N,), jnp.float32),
            pltpu.VMEM((N,), jnp.float32),
            pltpu.VMEM((K,), jnp.int32),
        ],
    )(pcs.reshape(-1))
    return out.reshape(B, K)
